# baseline (speedup 1.0000x reference)
import functools

import jax
import jax.numpy as jnp
from jax import lax
from jax.experimental import pallas as pl
from jax.experimental.pallas import tpu as pltpu
from jax.experimental.pallas import tpu_sc as plsc

D = 128
G = 256
NC, NS, LANES = 2, 16, 16
NW = NC * NS
CHUNK = 64
ATOM_SIZES = (119, 9, 11, 12, 9, 5, 8, 2, 2)
TCAT_ROWS = 256
ROWS_BLK = 1280


NB = 4


def _make_segsum(acc_rows, n_chunks, table_rows):
    assert n_chunks % (2 * NB) == 0 and n_chunks >= 2 * NB
    rpt = acc_rows // NS
    mesh = plsc.VectorSubcoreMesh(core_axis_name="c", subcore_axis_name="s")

    @functools.partial(
        pl.kernel, mesh=mesh,
        out_type=jax.ShapeDtypeStruct((NC, acc_rows, D), jnp.float32),
        scratch_types=(
            [pltpu.VMEM((2, CHUNK), jnp.int32)] * (2 * NB)
            + [pltpu.VMEM((CHUNK, D), jnp.float32)] * NB
            + [pltpu.VMEM_SHARED((acc_rows, D), jnp.float32)]
            + [pltpu.SemaphoreType.DMA] * (4 * NB)
        ),
    )
    def k(table_hbm, idx_hbm, zeros_hbm, out_hbm, *rest):
        ibufs = rest[:2 * NB]
        bufs = rest[2 * NB:3 * NB]
        acc = rest[3 * NB]
        sem_i = rest[3 * NB + 1:3 * NB + 1 + 2 * NB]
        sem_g = rest[3 * NB + 1 + 2 * NB:3 * NB + 1 + 3 * NB]
        sem_s = rest[3 * NB + 1 + 3 * NB:]
        cid = lax.axis_index("c")
        sid = lax.axis_index("s")
        wid = cid * NS + sid

        def idx_start(c, ib):
            pltpu.async_copy(idx_hbm.at[wid, c], ibufs[ib], sem_i[ib])

        def idx_wait(c, ib):
            pltpu.make_async_copy(
                idx_hbm.at[wid, c], ibufs[ib], sem_i[ib]).wait()

        def gat_start(c, ib, b):
            pltpu.async_copy(table_hbm.at[ibufs[ib].at[0]], bufs[b],
                             sem_g[b])

        def gat_wait(c, ib, b):
            pltpu.make_async_copy(table_hbm.at[ibufs[ib].at[0]], bufs[b],
                                  sem_g[b]).wait()

        def sca_start(c, ib, b):
            pltpu.async_copy(bufs[b], acc.at[ibufs[ib].at[1]], sem_s[b],
                             add=True)

        def sca_wait(c, ib, b):
            pltpu.make_async_copy(bufs[b], acc.at[ibufs[ib].at[1]],
                                  sem_s[b]).wait()

        pltpu.sync_copy(zeros_hbm, acc.at[pl.ds(sid * rpt, rpt)])
        plsc.subcore_barrier()

        for b in range(NB):
            idx_start(b, b)
        idx_wait(0, 0)
        gat_start(0, 0, 0)

        @pl.loop(0, n_chunks // (2 * NB))
        def _(i):
            for bb in range(2 * NB):
                j = i * (2 * NB) + bb
                b = bb % NB
                gat_wait(j, bb, b)
                sca_start(j, bb, b)
                ib_n = (bb + NB) % (2 * NB)

                @pl.when(j + NB < n_chunks)
                def _():
                    idx_start(j + NB, ib_n)

                bb1 = (bb + 1) % (2 * NB)
                b1 = (bb + 1) % NB
                ib_p = (bb + 1 - NB) % (2 * NB)

                @pl.when(j + 1 < n_chunks)
                def _():
                    idx_wait(j + 1, bb1)

                    @pl.when(j + 1 >= NB)
                    def _():
                        sca_wait(j + 1 - NB, ib_p, b1)
                    gat_start(j + 1, bb1, b1)

        for b in range(NB):
            c = n_chunks - NB + b
            sca_wait(c, c % (2 * NB), b)
        plsc.subcore_barrier()
        pltpu.sync_copy(acc.at[pl.ds(sid * rpt, rpt)],
                        out_hbm.at[cid, pl.ds(sid * rpt, rpt)])

    return k


def _make_hist(acc_rows, n_chunks):
    rpt = acc_rows // NS
    mesh = plsc.VectorSubcoreMesh(core_axis_name="c", subcore_axis_name="s")

    @functools.partial(
        pl.kernel, mesh=mesh,
        out_type=jax.ShapeDtypeStruct((NC, acc_rows, LANES), jnp.float32),
        scratch_types=[
            pltpu.VMEM((n_chunks, CHUNK), jnp.int32),
            pltpu.VMEM((CHUNK, LANES), jnp.float32),
            pltpu.VMEM_SHARED((acc_rows, LANES), jnp.float32),
        ],
    )
    def k(idx_hbm, ones_hbm, zeros_hbm, out_hbm, idxv, ones_v, acc):
        cid = lax.axis_index("c")
        sid = lax.axis_index("s")
        wid = cid * NS + sid
        pltpu.sync_copy(zeros_hbm, acc.at[pl.ds(sid * rpt, rpt)])
        pltpu.sync_copy(ones_hbm, ones_v)
        pltpu.sync_copy(idx_hbm.at[wid], idxv)
        plsc.subcore_barrier()

        @pl.loop(0, n_chunks)
        def _(j):
            pltpu.sync_copy(ones_v, acc.at[idxv.at[j]], add=True)

        plsc.subcore_barrier()
        pltpu.sync_copy(acc.at[pl.ds(sid * rpt, rpt)],
                        out_hbm.at[cid, pl.ds(sid * rpt, rpt)])

    return k


def _segsum_call(table, idx, zeros, acc_rows):
    n_chunks = idx.shape[1]
    return _make_segsum(acc_rows, n_chunks, table.shape[0])(
        table, idx, zeros)


def _hist_call(idx, ones, zeros, acc_rows):
    return _make_hist(acc_rows, idx.shape[1])(idx, ones, zeros)


def _dis_from(td):
    deg = td[0, :, 0:1] + td[1, :, 0:1] + 1.0
    return lax.rsqrt(deg)


def _enc_body(x_ref, tcat_ref, td_ref, o_ref):
    xv = x_ref[...]
    iota = lax.broadcasted_iota(jnp.int32, (ROWS_BLK, TCAT_ROWS), 1)
    m = jnp.zeros((ROWS_BLK, TCAT_ROWS), jnp.float32)
    off = 0
    for i, s in enumerate(ATOM_SIZES):
        m = m + (iota == (xv[:, i:i + 1] + off)).astype(jnp.float32)
        off += s
    h0 = jnp.dot(m, tcat_ref[...], preferred_element_type=jnp.float32)
    o_ref[...] = h0 * _dis_from(td_ref[...])


def _layer_body(t_ref, g_ref, td_ref, w_ref, b_ref, o_ref, *, last):
    t = t_ref[...]
    g = g_ref[...]
    dis = _dis_from(td_ref[...])
    u = (t[0] + t[1] + g) * dis
    v = jnp.dot(u, w_ref[...], preferred_element_type=jnp.float32) + b_ref[...]
    if last:
        o_ref[...] = v
    else:
        o_ref[...] = jnp.maximum(v, 0.0) * dis


def _final_body(s_ref, c_ref, w_ref, b_ref, lg_ref, lb_ref, o_ref):
    s = s_ref[...]
    c = c_ref[...]
    cnt = c[0, :, 0:1] + c[1, :, 0:1]
    pooled = (s[0] + s[1]) / jnp.maximum(cnt, 1.0)
    z = jnp.dot(pooled, w_ref[...], preferred_element_type=jnp.float32)
    z = z + b_ref[...]
    mu = jnp.mean(z, axis=-1, keepdims=True)
    var = jnp.mean((z - mu) ** 2, axis=-1, keepdims=True)
    o_ref[...] = (z - mu) * lax.rsqrt(var + 1e-5) * lg_ref[...] + lb_ref[...]


def kernel(x, edge_attr, edge_index, batch, atom_embs, bond_embs, Ws, bs,
           fc_W, fc_b, ln_g, ln_b):
    n = x.shape[0]
    e = edge_index.shape[1]
    npad = -(-n // ROWS_BLK) * ROWS_BLK
    nblk = npad // ROWS_BLK
    pad_rows = npad - n
    gacc = 2 * G

    step = NW * CHUNK * 2 * NB
    epad = -(-e // step) * step
    src = edge_index[0].astype(jnp.int32)
    dst = edge_index[1].astype(jnp.int32)
    pe = epad - e
    pad_i = (jnp.arange(pe, dtype=jnp.int32) % pad_rows) + n
    ec = epad // (NW * CHUNK)
    src_e = jnp.concatenate([src, pad_i]).reshape(NW, ec, 1, CHUNK)
    dst_e = jnp.concatenate([dst, pad_i]).reshape(NW, ec, 1, CHUNK)
    idx_e = jnp.concatenate([src_e, dst_e], axis=2)

    nppad = -(-n // step) * step
    pn = nppad - n
    pad_n = (jnp.arange(pn, dtype=jnp.int32) % pad_rows) + n
    pad_g = (jnp.arange(pn, dtype=jnp.int32) % G) + G
    pc = nppad // (NW * CHUNK)
    src_p = jnp.concatenate([jnp.arange(n, dtype=jnp.int32), pad_n]
                            ).reshape(NW, pc, 1, CHUNK)
    dst_p = jnp.concatenate([batch.astype(jnp.int32), pad_g]
                            ).reshape(NW, pc, 1, CHUNK)
    idx_p = jnp.concatenate([src_p, dst_p], axis=2)

    zer_n = jnp.zeros((npad // NS, D), jnp.float32)
    zer_g = jnp.zeros((gacc // NS, D), jnp.float32)
    zer_nh = jnp.zeros((npad // NS, LANES), jnp.float32)
    zer_gh = jnp.zeros((gacc // NS, LANES), jnp.float32)
    ones_h = jnp.ones((CHUNK, LANES), jnp.float32)

    xp = jnp.pad(x.astype(jnp.int32), ((0, npad - n), (0, 16 - x.shape[1])))
    tcat = jnp.concatenate(atom_embs, axis=0)
    tcat = jnp.pad(tcat, ((0, TCAT_ROWS - tcat.shape[0]), (0, 0)))

    td = _hist_call(dst_e.reshape(NW, ec, CHUNK), ones_h, zer_nh, npad)

    g = pl.pallas_call(
        _enc_body,
        grid=(nblk,),
        in_specs=[
            pl.BlockSpec((ROWS_BLK, 16), lambda i: (i, 0)),
            pl.BlockSpec((TCAT_ROWS, D), lambda i: (0, 0)),
            pl.BlockSpec((2, ROWS_BLK, LANES), lambda i: (0, i, 0)),
        ],
        out_specs=pl.BlockSpec((ROWS_BLK, D), lambda i: (i, 0)),
        out_shape=jax.ShapeDtypeStruct((npad, D), jnp.float32),
    )(xp, tcat, td)

    for l in range(len(Ws)):
        t = _segsum_call(g, idx_e, zer_n, npad)
        g = pl.pallas_call(
            functools.partial(_layer_body, last=(l == len(Ws) - 1)),
            grid=(nblk,),
            in_specs=[
                pl.BlockSpec((2, ROWS_BLK, D), lambda i: (0, i, 0)),
                pl.BlockSpec((ROWS_BLK, D), lambda i: (i, 0)),
                pl.BlockSpec((2, ROWS_BLK, LANES), lambda i: (0, i, 0)),
                pl.BlockSpec((D, D), lambda i: (0, 0)),
                pl.BlockSpec((1, D), lambda i: (0, 0)),
            ],
            out_specs=pl.BlockSpec((ROWS_BLK, D), lambda i: (i, 0)),
            out_shape=jax.ShapeDtypeStruct((npad, D), jnp.float32),
        )(t, g, td, Ws[l], bs[l].reshape(1, D))

    s = _segsum_call(g, idx_p, zer_g, gacc)
    c = _hist_call(dst_p.reshape(NW, pc, CHUNK), ones_h, zer_gh, gacc)

    z = pl.pallas_call(
        _final_body,
        grid=(1,),
        in_specs=[
            pl.BlockSpec((2, G, D), lambda i: (0, 0, 0)),
            pl.BlockSpec((2, G, LANES), lambda i: (0, 0, 0)),
            pl.BlockSpec((D, D), lambda i: (0, 0)),
            pl.BlockSpec((1, D), lambda i: (0, 0)),
            pl.BlockSpec((1, D), lambda i: (0, 0)),
            pl.BlockSpec((1, D), lambda i: (0, 0)),
        ],
        out_specs=pl.BlockSpec((G, D), lambda i: (0, 0)),
        out_shape=jax.ShapeDtypeStruct((G, D), jnp.float32),
    )(s, c, fc_W, fc_b.reshape(1, D), ln_g.reshape(1, D), ln_b.reshape(1, D))
    return z


# speedup vs baseline: 1.3012x; 1.3012x over previous
import functools

import jax
import jax.numpy as jnp
from jax import lax
from jax.experimental import pallas as pl
from jax.experimental.pallas import tpu as pltpu
from jax.experimental.pallas import tpu_sc as plsc

D = 128
G = 256
NC, NS, LANES = 2, 16, 16
NW = NC * NS
CHUNK = 128
ATOM_SIZES = (119, 9, 11, 12, 9, 5, 8, 2, 2)
TCAT_ROWS = 256
ROWS_BLK = 1280


NB = 2


def _make_segsum(acc_rows, n_chunks, table_rows):
    assert n_chunks % (2 * NB) == 0 and n_chunks >= 2 * NB
    rpt = acc_rows // NS
    mesh = plsc.VectorSubcoreMesh(core_axis_name="c", subcore_axis_name="s")

    @functools.partial(
        pl.kernel, mesh=mesh,
        out_type=jax.ShapeDtypeStruct((NC, acc_rows, D), jnp.float32),
        scratch_types=(
            [pltpu.VMEM((2, CHUNK), jnp.int32)] * (2 * NB)
            + [pltpu.VMEM((CHUNK, D), jnp.float32)] * NB
            + [pltpu.VMEM_SHARED((acc_rows, D), jnp.float32)]
            + [pltpu.SemaphoreType.DMA] * (4 * NB)
        ),
    )
    def k(table_hbm, idx_hbm, zeros_hbm, out_hbm, *rest):
        ibufs = rest[:2 * NB]
        bufs = rest[2 * NB:3 * NB]
        acc = rest[3 * NB]
        sem_i = rest[3 * NB + 1:3 * NB + 1 + 2 * NB]
        sem_g = rest[3 * NB + 1 + 2 * NB:3 * NB + 1 + 3 * NB]
        sem_s = rest[3 * NB + 1 + 3 * NB:]
        cid = lax.axis_index("c")
        sid = lax.axis_index("s")
        wid = cid * NS + sid

        def idx_start(c, ib):
            pltpu.async_copy(idx_hbm.at[wid, c], ibufs[ib], sem_i[ib])

        def idx_wait(c, ib):
            pltpu.make_async_copy(
                idx_hbm.at[wid, c], ibufs[ib], sem_i[ib]).wait()

        def gat_start(c, ib, b):
            pltpu.async_copy(table_hbm.at[ibufs[ib].at[0]], bufs[b],
                             sem_g[b])

        def gat_wait(c, ib, b):
            pltpu.make_async_copy(table_hbm.at[ibufs[ib].at[0]], bufs[b],
                                  sem_g[b]).wait()

        def sca_start(c, ib, b):
            pltpu.async_copy(bufs[b], acc.at[ibufs[ib].at[1]], sem_s[b],
                             add=True)

        def sca_wait(c, ib, b):
            pltpu.make_async_copy(bufs[b], acc.at[ibufs[ib].at[1]],
                                  sem_s[b]).wait()

        pltpu.sync_copy(zeros_hbm, acc.at[pl.ds(sid * rpt, rpt)])
        plsc.subcore_barrier()

        for b in range(NB):
            idx_start(b, b)
        idx_wait(0, 0)
        gat_start(0, 0, 0)

        @pl.loop(0, n_chunks // (2 * NB))
        def _(i):
            for bb in range(2 * NB):
                j = i * (2 * NB) + bb
                b = bb % NB
                gat_wait(j, bb, b)
                sca_start(j, bb, b)
                ib_n = (bb + NB) % (2 * NB)

                @pl.when(j + NB < n_chunks)
                def _():
                    idx_start(j + NB, ib_n)

                bb1 = (bb + 1) % (2 * NB)
                b1 = (bb + 1) % NB
                ib_p = (bb + 1 - NB) % (2 * NB)

                @pl.when(j + 1 < n_chunks)
                def _():
                    idx_wait(j + 1, bb1)

                    @pl.when(j + 1 >= NB)
                    def _():
                        sca_wait(j + 1 - NB, ib_p, b1)
                    gat_start(j + 1, bb1, b1)

        for b in range(NB):
            c = n_chunks - NB + b
            sca_wait(c, c % (2 * NB), b)
        plsc.subcore_barrier()
        pltpu.sync_copy(acc.at[pl.ds(sid * rpt, rpt)],
                        out_hbm.at[cid, pl.ds(sid * rpt, rpt)])

    return k


def _make_hist(acc_rows, n_chunks):
    rpt = acc_rows // NS
    mesh = plsc.VectorSubcoreMesh(core_axis_name="c", subcore_axis_name="s")

    @functools.partial(
        pl.kernel, mesh=mesh,
        out_type=jax.ShapeDtypeStruct((NC, acc_rows, LANES), jnp.float32),
        scratch_types=[
            pltpu.VMEM((n_chunks, CHUNK), jnp.int32),
            pltpu.VMEM((CHUNK, LANES), jnp.float32),
            pltpu.VMEM_SHARED((acc_rows, LANES), jnp.float32),
        ],
    )
    def k(idx_hbm, ones_hbm, zeros_hbm, out_hbm, idxv, ones_v, acc):
        cid = lax.axis_index("c")
        sid = lax.axis_index("s")
        wid = cid * NS + sid
        pltpu.sync_copy(zeros_hbm, acc.at[pl.ds(sid * rpt, rpt)])
        pltpu.sync_copy(ones_hbm, ones_v)
        pltpu.sync_copy(idx_hbm.at[wid], idxv)
        plsc.subcore_barrier()

        @pl.loop(0, n_chunks)
        def _(j):
            pltpu.sync_copy(ones_v, acc.at[idxv.at[j]], add=True)

        plsc.subcore_barrier()
        pltpu.sync_copy(acc.at[pl.ds(sid * rpt, rpt)],
                        out_hbm.at[cid, pl.ds(sid * rpt, rpt)])

    return k


def _segsum_call(table, idx, zeros, acc_rows):
    n_chunks = idx.shape[1]
    return _make_segsum(acc_rows, n_chunks, table.shape[0])(
        table, idx, zeros)


def _hist_call(idx, ones, zeros, acc_rows):
    return _make_hist(acc_rows, idx.shape[1])(idx, ones, zeros)


def _dis_from(td):
    deg = td[0, :, 0:1] + td[1, :, 0:1] + 1.0
    return lax.rsqrt(deg)


def _enc_body(x_ref, tcat_ref, td_ref, o_ref):
    xv = x_ref[...]
    iota = lax.broadcasted_iota(jnp.int32, (ROWS_BLK, TCAT_ROWS), 1)
    m = jnp.zeros((ROWS_BLK, TCAT_ROWS), jnp.float32)
    off = 0
    for i, s in enumerate(ATOM_SIZES):
        m = m + (iota == (xv[:, i:i + 1] + off)).astype(jnp.float32)
        off += s
    h0 = jnp.dot(m, tcat_ref[...], preferred_element_type=jnp.float32)
    o_ref[...] = h0 * _dis_from(td_ref[...])


def _layer_body(t_ref, g_ref, td_ref, w_ref, b_ref, o_ref, *, last):
    t = t_ref[...]
    g = g_ref[...]
    dis = _dis_from(td_ref[...])
    u = (t[0] + t[1] + g) * dis
    v = jnp.dot(u, w_ref[...], preferred_element_type=jnp.float32) + b_ref[...]
    if last:
        o_ref[...] = v
    else:
        o_ref[...] = jnp.maximum(v, 0.0) * dis


def _final_body(s_ref, c_ref, w_ref, b_ref, lg_ref, lb_ref, o_ref):
    s = s_ref[...]
    c = c_ref[...]
    cnt = c[0, :, 0:1] + c[1, :, 0:1]
    pooled = (s[0] + s[1]) / jnp.maximum(cnt, 1.0)
    z = jnp.dot(pooled, w_ref[...], preferred_element_type=jnp.float32)
    z = z + b_ref[...]
    mu = jnp.mean(z, axis=-1, keepdims=True)
    var = jnp.mean((z - mu) ** 2, axis=-1, keepdims=True)
    o_ref[...] = (z - mu) * lax.rsqrt(var + 1e-5) * lg_ref[...] + lb_ref[...]


def kernel(x, edge_attr, edge_index, batch, atom_embs, bond_embs, Ws, bs,
           fc_W, fc_b, ln_g, ln_b):
    n = x.shape[0]
    e = edge_index.shape[1]
    npad = -(-n // ROWS_BLK) * ROWS_BLK
    nblk = npad // ROWS_BLK
    pad_rows = npad - n
    gacc = 2 * G

    step = NW * CHUNK * 2 * NB
    epad = -(-e // step) * step
    src = edge_index[0].astype(jnp.int32)
    dst = edge_index[1].astype(jnp.int32)
    pe = epad - e
    pad_i = (jnp.arange(pe, dtype=jnp.int32) % pad_rows) + n
    ec = epad // (NW * CHUNK)
    src_e = jnp.concatenate([src, pad_i]).reshape(NW, ec, 1, CHUNK)
    dst_e = jnp.concatenate([dst, pad_i]).reshape(NW, ec, 1, CHUNK)
    idx_e = jnp.concatenate([src_e, dst_e], axis=2)

    nppad = -(-n // step) * step
    pn = nppad - n
    pad_n = (jnp.arange(pn, dtype=jnp.int32) % pad_rows) + n
    pad_g = (jnp.arange(pn, dtype=jnp.int32) % G) + G
    pc = nppad // (NW * CHUNK)
    src_p = jnp.concatenate([jnp.arange(n, dtype=jnp.int32), pad_n]
                            ).reshape(NW, pc, 1, CHUNK)
    dst_p = jnp.concatenate([batch.astype(jnp.int32), pad_g]
                            ).reshape(NW, pc, 1, CHUNK)
    idx_p = jnp.concatenate([src_p, dst_p], axis=2)

    zer_n = jnp.zeros((npad // NS, D), jnp.float32)
    zer_g = jnp.zeros((gacc // NS, D), jnp.float32)
    zer_nh = jnp.zeros((npad // NS, LANES), jnp.float32)
    zer_gh = jnp.zeros((gacc // NS, LANES), jnp.float32)
    ones_h = jnp.ones((CHUNK, LANES), jnp.float32)

    xp = jnp.pad(x.astype(jnp.int32), ((0, npad - n), (0, 16 - x.shape[1])))
    tcat = jnp.concatenate(atom_embs, axis=0)
    tcat = jnp.pad(tcat, ((0, TCAT_ROWS - tcat.shape[0]), (0, 0)))

    td = _hist_call(dst_e.reshape(NW, ec, CHUNK), ones_h, zer_nh, npad)

    g = pl.pallas_call(
        _enc_body,
        grid=(nblk,),
        in_specs=[
            pl.BlockSpec((ROWS_BLK, 16), lambda i: (i, 0)),
            pl.BlockSpec((TCAT_ROWS, D), lambda i: (0, 0)),
            pl.BlockSpec((2, ROWS_BLK, LANES), lambda i: (0, i, 0)),
        ],
        out_specs=pl.BlockSpec((ROWS_BLK, D), lambda i: (i, 0)),
        out_shape=jax.ShapeDtypeStruct((npad, D), jnp.float32),
    )(xp, tcat, td)

    for l in range(len(Ws)):
        t = _segsum_call(g, idx_e, zer_n, npad)
        g = pl.pallas_call(
            functools.partial(_layer_body, last=(l == len(Ws) - 1)),
            grid=(nblk,),
            in_specs=[
                pl.BlockSpec((2, ROWS_BLK, D), lambda i: (0, i, 0)),
                pl.BlockSpec((ROWS_BLK, D), lambda i: (i, 0)),
                pl.BlockSpec((2, ROWS_BLK, LANES), lambda i: (0, i, 0)),
                pl.BlockSpec((D, D), lambda i: (0, 0)),
                pl.BlockSpec((1, D), lambda i: (0, 0)),
            ],
            out_specs=pl.BlockSpec((ROWS_BLK, D), lambda i: (i, 0)),
            out_shape=jax.ShapeDtypeStruct((npad, D), jnp.float32),
        )(t, g, td, Ws[l], bs[l].reshape(1, D))

    s = _segsum_call(g, idx_p, zer_g, gacc)
    c = _hist_call(dst_p.reshape(NW, pc, CHUNK), ones_h, zer_gh, gacc)

    z = pl.pallas_call(
        _final_body,
        grid=(1,),
        in_specs=[
            pl.BlockSpec((2, G, D), lambda i: (0, 0, 0)),
            pl.BlockSpec((2, G, LANES), lambda i: (0, 0, 0)),
            pl.BlockSpec((D, D), lambda i: (0, 0)),
            pl.BlockSpec((1, D), lambda i: (0, 0)),
            pl.BlockSpec((1, D), lambda i: (0, 0)),
            pl.BlockSpec((1, D), lambda i: (0, 0)),
        ],
        out_specs=pl.BlockSpec((G, D), lambda i: (0, 0)),
        out_shape=jax.ShapeDtypeStruct((G, D), jnp.float32),
    )(s, c, fc_W, fc_b.reshape(1, D), ln_g.reshape(1, D), ln_b.reshape(1, D))
    return z


# speedup vs baseline: 1.3259x; 1.0190x over previous
import functools

import jax
import jax.numpy as jnp
from jax import lax
from jax.experimental import pallas as pl
from jax.experimental.pallas import tpu as pltpu
from jax.experimental.pallas import tpu_sc as plsc

D = 128
G = 256
NC, NS, LANES = 2, 16, 16
NW = NC * NS
CHUNK = 128
ATOM_SIZES = (119, 9, 11, 12, 9, 5, 8, 2, 2)
TCAT_ROWS = 256
ROWS_BLK = 1280


NB = 2


def _make_segsum(acc_rows, n_chunks, table_rows):
    assert n_chunks % (2 * NB) == 0 and n_chunks >= 2 * NB
    rpt = acc_rows // NS
    mesh = plsc.VectorSubcoreMesh(core_axis_name="c", subcore_axis_name="s")

    @functools.partial(
        pl.kernel, mesh=mesh,
        out_type=jax.ShapeDtypeStruct((NC, acc_rows, D), jnp.float32),
        scratch_types=(
            [pltpu.VMEM((2, CHUNK), jnp.int32)] * (2 * NB)
            + [pltpu.VMEM((CHUNK, D), jnp.float32)] * NB
            + [pltpu.VMEM_SHARED((acc_rows, D), jnp.float32)]
            + [pltpu.SemaphoreType.DMA] * (4 * NB)
        ),
    )
    def k(table_hbm, idx_hbm, zeros_hbm, out_hbm, *rest):
        ibufs = rest[:2 * NB]
        bufs = rest[2 * NB:3 * NB]
        acc = rest[3 * NB]
        sem_i = rest[3 * NB + 1:3 * NB + 1 + 2 * NB]
        sem_g = rest[3 * NB + 1 + 2 * NB:3 * NB + 1 + 3 * NB]
        sem_s = rest[3 * NB + 1 + 3 * NB:]
        cid = lax.axis_index("c")
        sid = lax.axis_index("s")
        wid = cid * NS + sid

        def idx_start(c, ib):
            pltpu.async_copy(idx_hbm.at[wid, c], ibufs[ib], sem_i[ib])

        def idx_wait(c, ib):
            pltpu.make_async_copy(
                idx_hbm.at[wid, c], ibufs[ib], sem_i[ib]).wait()

        def gat_start(c, ib, b):
            pltpu.async_copy(table_hbm.at[ibufs[ib].at[0]], bufs[b],
                             sem_g[b])

        def gat_wait(c, ib, b):
            pltpu.make_async_copy(table_hbm.at[ibufs[ib].at[0]], bufs[b],
                                  sem_g[b]).wait()

        def sca_start(c, ib, b):
            pltpu.async_copy(bufs[b], acc.at[ibufs[ib].at[1]], sem_s[b],
                             add=True)

        def sca_wait(c, ib, b):
            pltpu.make_async_copy(bufs[b], acc.at[ibufs[ib].at[1]],
                                  sem_s[b]).wait()

        pltpu.sync_copy(zeros_hbm, acc.at[pl.ds(sid * rpt, rpt)])
        plsc.subcore_barrier()

        for b in range(NB):
            idx_start(b, b)
        idx_wait(0, 0)
        gat_start(0, 0, 0)

        @pl.loop(0, n_chunks // (2 * NB))
        def _(i):
            for bb in range(2 * NB):
                j = i * (2 * NB) + bb
                b = bb % NB
                gat_wait(j, bb, b)
                sca_start(j, bb, b)
                ib_n = (bb + NB) % (2 * NB)

                @pl.when(j + NB < n_chunks)
                def _():
                    idx_start(j + NB, ib_n)

                bb1 = (bb + 1) % (2 * NB)
                b1 = (bb + 1) % NB
                ib_p = (bb + 1 - NB) % (2 * NB)

                @pl.when(j + 1 < n_chunks)
                def _():
                    idx_wait(j + 1, bb1)

                    @pl.when(j + 1 >= NB)
                    def _():
                        sca_wait(j + 1 - NB, ib_p, b1)
                    gat_start(j + 1, bb1, b1)

        for b in range(NB):
            c = n_chunks - NB + b
            sca_wait(c, c % (2 * NB), b)
        plsc.subcore_barrier()
        pltpu.sync_copy(acc.at[pl.ds(sid * rpt, rpt)],
                        out_hbm.at[cid, pl.ds(sid * rpt, rpt)])

    return k


def _make_hist(acc_rows, n_chunks):
    rpt = acc_rows // NS
    mesh = plsc.VectorSubcoreMesh(core_axis_name="c", subcore_axis_name="s")

    @functools.partial(
        pl.kernel, mesh=mesh,
        out_type=jax.ShapeDtypeStruct((NC, acc_rows, LANES), jnp.float32),
        scratch_types=[
            pltpu.VMEM((n_chunks, CHUNK), jnp.int32),
            pltpu.VMEM((CHUNK, LANES), jnp.float32),
            pltpu.VMEM_SHARED((acc_rows, LANES), jnp.float32),
        ],
    )
    def k(idx_hbm, ones_hbm, zeros_hbm, out_hbm, idxv, ones_v, acc):
        cid = lax.axis_index("c")
        sid = lax.axis_index("s")
        wid = cid * NS + sid
        pltpu.sync_copy(zeros_hbm, acc.at[pl.ds(sid * rpt, rpt)])
        pltpu.sync_copy(ones_hbm, ones_v)
        pltpu.sync_copy(idx_hbm.at[wid], idxv)
        plsc.subcore_barrier()

        @pl.loop(0, n_chunks)
        def _(j):
            pltpu.sync_copy(ones_v, acc.at[idxv.at[j]], add=True)

        plsc.subcore_barrier()
        pltpu.sync_copy(acc.at[pl.ds(sid * rpt, rpt)],
                        out_hbm.at[cid, pl.ds(sid * rpt, rpt)])

    return k


def _segsum_call(table, idx, zeros, acc_rows):
    n_chunks = idx.shape[1]
    return _make_segsum(acc_rows, n_chunks, table.shape[0])(
        table, idx, zeros)


def _hist_call(idx, ones, zeros, acc_rows):
    return _make_hist(acc_rows, idx.shape[1])(idx, ones, zeros)


def _dis_from(td):
    deg = td[0, :, 0:1] + td[1, :, 0:1] + 1.0
    return lax.rsqrt(deg)


def _enc_body(x_ref, tcat_ref, td_ref, o_ref):
    xv = x_ref[...]
    iota = lax.broadcasted_iota(jnp.int32, (ROWS_BLK, TCAT_ROWS), 1)
    m = jnp.zeros((ROWS_BLK, TCAT_ROWS), jnp.float32)
    off = 0
    for i, s in enumerate(ATOM_SIZES):
        m = m + (iota == (xv[:, i:i + 1] + off)).astype(jnp.float32)
        off += s
    h0 = jnp.dot(m, tcat_ref[...], preferred_element_type=jnp.float32)
    o_ref[...] = h0 * _dis_from(td_ref[...])


def _layer_body(t_ref, g_ref, td_ref, w_ref, b_ref, o_ref, *, last):
    t = t_ref[...]
    g = g_ref[...]
    dis = _dis_from(td_ref[...])
    u = (t[0] + t[1] + g) * dis
    v = jnp.dot(u, w_ref[...], preferred_element_type=jnp.float32) + b_ref[...]
    if last:
        o_ref[...] = v
    else:
        o_ref[...] = jnp.maximum(v, 0.0) * dis


def _final_body(h_ref, batch_ref, w_ref, b_ref, lg_ref, lb_ref, o_ref):
    npad = h_ref.shape[0]
    oh = (lax.broadcasted_iota(jnp.int32, (G, npad), 0)
          == batch_ref[...]).astype(jnp.float32)
    s = jnp.dot(oh, h_ref[...], preferred_element_type=jnp.float32)
    cnt = jnp.sum(oh, axis=1, keepdims=True)
    pooled = s / jnp.maximum(cnt, 1.0)
    z = jnp.dot(pooled, w_ref[...], preferred_element_type=jnp.float32)
    z = z + b_ref[...]
    mu = jnp.mean(z, axis=-1, keepdims=True)
    var = jnp.mean((z - mu) ** 2, axis=-1, keepdims=True)
    o_ref[...] = (z - mu) * lax.rsqrt(var + 1e-5) * lg_ref[...] + lb_ref[...]


def kernel(x, edge_attr, edge_index, batch, atom_embs, bond_embs, Ws, bs,
           fc_W, fc_b, ln_g, ln_b):
    n = x.shape[0]
    e = edge_index.shape[1]
    npad = -(-n // ROWS_BLK) * ROWS_BLK
    nblk = npad // ROWS_BLK
    pad_rows = npad - n

    step = NW * CHUNK * 2 * NB
    epad = -(-e // step) * step
    src = edge_index[0].astype(jnp.int32)
    dst = edge_index[1].astype(jnp.int32)
    pe = epad - e
    pad_i = (jnp.arange(pe, dtype=jnp.int32) % pad_rows) + n
    ec = epad // (NW * CHUNK)
    src_e = jnp.concatenate([src, pad_i]).reshape(NW, ec, 1, CHUNK)
    dst_e = jnp.concatenate([dst, pad_i]).reshape(NW, ec, 1, CHUNK)
    idx_e = jnp.concatenate([src_e, dst_e], axis=2)

    bpad = jnp.pad(batch.astype(jnp.int32), (0, npad - n),
                   constant_values=G).reshape(1, npad)

    zer_n = jnp.zeros((npad // NS, D), jnp.float32)
    zer_nh = jnp.zeros((npad // NS, LANES), jnp.float32)
    ones_h = jnp.ones((CHUNK, LANES), jnp.float32)

    xp = jnp.pad(x.astype(jnp.int32), ((0, npad - n), (0, 16 - x.shape[1])))
    tcat = jnp.concatenate(atom_embs, axis=0)
    tcat = jnp.pad(tcat, ((0, TCAT_ROWS - tcat.shape[0]), (0, 0)))

    td = _hist_call(dst_e.reshape(NW, ec, CHUNK), ones_h, zer_nh, npad)

    g = pl.pallas_call(
        _enc_body,
        grid=(nblk,),
        in_specs=[
            pl.BlockSpec((ROWS_BLK, 16), lambda i: (i, 0)),
            pl.BlockSpec((TCAT_ROWS, D), lambda i: (0, 0)),
            pl.BlockSpec((2, ROWS_BLK, LANES), lambda i: (0, i, 0)),
        ],
        out_specs=pl.BlockSpec((ROWS_BLK, D), lambda i: (i, 0)),
        out_shape=jax.ShapeDtypeStruct((npad, D), jnp.float32),
    )(xp, tcat, td)

    for l in range(len(Ws)):
        t = _segsum_call(g, idx_e, zer_n, npad)
        g = pl.pallas_call(
            functools.partial(_layer_body, last=(l == len(Ws) - 1)),
            grid=(nblk,),
            in_specs=[
                pl.BlockSpec((2, ROWS_BLK, D), lambda i: (0, i, 0)),
                pl.BlockSpec((ROWS_BLK, D), lambda i: (i, 0)),
                pl.BlockSpec((2, ROWS_BLK, LANES), lambda i: (0, i, 0)),
                pl.BlockSpec((D, D), lambda i: (0, 0)),
                pl.BlockSpec((1, D), lambda i: (0, 0)),
            ],
            out_specs=pl.BlockSpec((ROWS_BLK, D), lambda i: (i, 0)),
            out_shape=jax.ShapeDtypeStruct((npad, D), jnp.float32),
        )(t, g, td, Ws[l], bs[l].reshape(1, D))

    z = pl.pallas_call(
        _final_body,
        grid=(1,),
        in_specs=[
            pl.BlockSpec((npad, D), lambda i: (0, 0)),
            pl.BlockSpec((1, npad), lambda i: (0, 0)),
            pl.BlockSpec((D, D), lambda i: (0, 0)),
            pl.BlockSpec((1, D), lambda i: (0, 0)),
            pl.BlockSpec((1, D), lambda i: (0, 0)),
            pl.BlockSpec((1, D), lambda i: (0, 0)),
        ],
        out_specs=pl.BlockSpec((G, D), lambda i: (0, 0)),
        out_shape=jax.ShapeDtypeStruct((G, D), jnp.float32),
    )(g, bpad, fc_W, fc_b.reshape(1, D), ln_g.reshape(1, D), ln_b.reshape(1, D))
    return z


# speedup vs baseline: 1.3405x; 1.0110x over previous
import functools

import jax
import jax.numpy as jnp
from jax import lax
from jax.experimental import pallas as pl
from jax.experimental.pallas import tpu as pltpu
from jax.experimental.pallas import tpu_sc as plsc

D = 128
G = 256
NC, NS, LANES = 2, 16, 16
NW = NC * NS
CHUNK = 128
ATOM_SIZES = (119, 9, 11, 12, 9, 5, 8, 2, 2)
TCAT_ROWS = 256
ROWS_BLK = 1280


NB = 2


def _make_segsum(acc_rows, n_chunks, table_rows):
    assert n_chunks % (2 * NB) == 0 and n_chunks >= 2 * NB
    rpt = acc_rows // NS
    mesh = plsc.VectorSubcoreMesh(core_axis_name="c", subcore_axis_name="s")

    @functools.partial(
        pl.kernel, mesh=mesh,
        out_type=jax.ShapeDtypeStruct((NC, acc_rows, D), jnp.float32),
        scratch_types=(
            [pltpu.VMEM((2, CHUNK), jnp.int32)] * (2 * NB)
            + [pltpu.VMEM((CHUNK, D), jnp.float32)] * NB
            + [pltpu.VMEM_SHARED((acc_rows, D), jnp.float32)]
            + [pltpu.SemaphoreType.DMA] * (4 * NB)
        ),
    )
    def k(table_hbm, idx_hbm, zeros_hbm, out_hbm, *rest):
        ibufs = rest[:2 * NB]
        bufs = rest[2 * NB:3 * NB]
        acc = rest[3 * NB]
        sem_i = rest[3 * NB + 1:3 * NB + 1 + 2 * NB]
        sem_g = rest[3 * NB + 1 + 2 * NB:3 * NB + 1 + 3 * NB]
        sem_s = rest[3 * NB + 1 + 3 * NB:]
        cid = lax.axis_index("c")
        sid = lax.axis_index("s")
        wid = cid * NS + sid

        def idx_start(c, ib):
            pltpu.async_copy(idx_hbm.at[wid, c], ibufs[ib], sem_i[ib])

        def idx_wait(c, ib):
            pltpu.make_async_copy(
                idx_hbm.at[wid, c], ibufs[ib], sem_i[ib]).wait()

        def gat_start(c, ib, b):
            pltpu.async_copy(table_hbm.at[ibufs[ib].at[0]], bufs[b],
                             sem_g[b])

        def gat_wait(c, ib, b):
            pltpu.make_async_copy(table_hbm.at[ibufs[ib].at[0]], bufs[b],
                                  sem_g[b]).wait()

        def sca_start(c, ib, b):
            pltpu.async_copy(bufs[b], acc.at[ibufs[ib].at[1]], sem_s[b],
                             add=True)

        def sca_wait(c, ib, b):
            pltpu.make_async_copy(bufs[b], acc.at[ibufs[ib].at[1]],
                                  sem_s[b]).wait()

        for b in range(NB):
            idx_start(b, b)
        idx_wait(0, 0)
        gat_start(0, 0, 0)
        pltpu.sync_copy(zeros_hbm, acc.at[pl.ds(sid * rpt, rpt)])
        plsc.subcore_barrier()

        @pl.loop(0, n_chunks // (2 * NB))
        def _(i):
            for bb in range(2 * NB):
                j = i * (2 * NB) + bb
                b = bb % NB
                gat_wait(j, bb, b)
                sca_start(j, bb, b)
                ib_n = (bb + NB) % (2 * NB)

                @pl.when(j + NB < n_chunks)
                def _():
                    idx_start(j + NB, ib_n)

                bb1 = (bb + 1) % (2 * NB)
                b1 = (bb + 1) % NB
                ib_p = (bb + 1 - NB) % (2 * NB)

                @pl.when(j + 1 < n_chunks)
                def _():
                    idx_wait(j + 1, bb1)

                    @pl.when(j + 1 >= NB)
                    def _():
                        sca_wait(j + 1 - NB, ib_p, b1)
                    gat_start(j + 1, bb1, b1)

        for b in range(NB):
            c = n_chunks - NB + b
            sca_wait(c, c % (2 * NB), b)
        plsc.subcore_barrier()
        pltpu.sync_copy(acc.at[pl.ds(sid * rpt, rpt)],
                        out_hbm.at[cid, pl.ds(sid * rpt, rpt)])

    return k


def _make_hist(acc_rows, n_chunks):
    rpt = acc_rows // NS
    KH = 8
    assert n_chunks % KH == 0
    mesh = plsc.VectorSubcoreMesh(core_axis_name="c", subcore_axis_name="s")

    @functools.partial(
        pl.kernel, mesh=mesh,
        out_type=jax.ShapeDtypeStruct((NC, acc_rows, LANES), jnp.float32),
        scratch_types=(
            [pltpu.VMEM((n_chunks, CHUNK), jnp.int32),
             pltpu.VMEM((CHUNK, LANES), jnp.float32),
             pltpu.VMEM_SHARED((acc_rows, LANES), jnp.float32)]
            + [pltpu.SemaphoreType.DMA] * (KH + 1)
        ),
    )
    def k(idx_hbm, ones_hbm, zeros_hbm, out_hbm, idxv, ones_v, acc, *sems):
        sem_x = sems[KH]
        cid = lax.axis_index("c")
        sid = lax.axis_index("s")
        wid = cid * NS + sid

        def sca_wait(c, b):
            pltpu.make_async_copy(ones_v, acc.at[idxv.at[c]],
                                  sems[b]).wait()

        pltpu.async_copy(idx_hbm.at[wid], idxv, sem_x)
        pltpu.sync_copy(ones_hbm, ones_v)
        pltpu.sync_copy(zeros_hbm, acc.at[pl.ds(sid * rpt, rpt)])
        pltpu.make_async_copy(idx_hbm.at[wid], idxv, sem_x).wait()
        plsc.subcore_barrier()

        @pl.loop(0, n_chunks // KH)
        def _(i):
            for b in range(KH):
                j = i * KH + b

                @pl.when(j >= KH)
                def _():
                    sca_wait(j - KH, b)
                pltpu.async_copy(ones_v, acc.at[idxv.at[j]], sems[b],
                                 add=True)

        for b in range(KH):
            sca_wait(n_chunks - KH + b, b)
        plsc.subcore_barrier()
        pltpu.sync_copy(acc.at[pl.ds(sid * rpt, rpt)],
                        out_hbm.at[cid, pl.ds(sid * rpt, rpt)])

    return k


def _segsum_call(table, idx, zeros, acc_rows):
    n_chunks = idx.shape[1]
    return _make_segsum(acc_rows, n_chunks, table.shape[0])(
        table, idx, zeros)


def _hist_call(idx, ones, zeros, acc_rows):
    return _make_hist(acc_rows, idx.shape[1])(idx, ones, zeros)


def _dis_from(td):
    deg = td[0, :, 0:1] + td[1, :, 0:1] + 1.0
    return lax.rsqrt(deg)


def _enc_body(x_ref, tcat_ref, td_ref, o_ref):
    xv = x_ref[...]
    iota = lax.broadcasted_iota(jnp.int32, (ROWS_BLK, TCAT_ROWS), 1)
    m = jnp.zeros((ROWS_BLK, TCAT_ROWS), jnp.float32)
    off = 0
    for i, s in enumerate(ATOM_SIZES):
        m = m + (iota == (xv[:, i:i + 1] + off)).astype(jnp.float32)
        off += s
    h0 = jnp.dot(m, tcat_ref[...], preferred_element_type=jnp.float32)
    o_ref[...] = h0 * _dis_from(td_ref[...])


def _layer_body(t_ref, g_ref, td_ref, w_ref, b_ref, o_ref, *, last):
    t = t_ref[...]
    g = g_ref[...]
    dis = _dis_from(td_ref[...])
    u = (t[0] + t[1] + g) * dis
    v = jnp.dot(u, w_ref[...], preferred_element_type=jnp.float32) + b_ref[...]
    if last:
        o_ref[...] = v
    else:
        o_ref[...] = jnp.maximum(v, 0.0) * dis


def _final_body(h_ref, batch_ref, w_ref, b_ref, lg_ref, lb_ref, o_ref):
    npad = h_ref.shape[0]
    oh = (lax.broadcasted_iota(jnp.int32, (G, npad), 0)
          == batch_ref[...]).astype(jnp.float32)
    s = jnp.dot(oh, h_ref[...], preferred_element_type=jnp.float32)
    cnt = jnp.sum(oh, axis=1, keepdims=True)
    pooled = s / jnp.maximum(cnt, 1.0)
    z = jnp.dot(pooled, w_ref[...], preferred_element_type=jnp.float32)
    z = z + b_ref[...]
    mu = jnp.mean(z, axis=-1, keepdims=True)
    var = jnp.mean((z - mu) ** 2, axis=-1, keepdims=True)
    o_ref[...] = (z - mu) * lax.rsqrt(var + 1e-5) * lg_ref[...] + lb_ref[...]


def kernel(x, edge_attr, edge_index, batch, atom_embs, bond_embs, Ws, bs,
           fc_W, fc_b, ln_g, ln_b):
    n = x.shape[0]
    e = edge_index.shape[1]
    npad = -(-n // ROWS_BLK) * ROWS_BLK
    nblk = npad // ROWS_BLK
    pad_rows = npad - n

    step = NW * CHUNK * 2 * NB
    epad = -(-e // step) * step
    src = edge_index[0].astype(jnp.int32)
    dst = edge_index[1].astype(jnp.int32)
    pe = epad - e
    pad_i = (jnp.arange(pe, dtype=jnp.int32) % pad_rows) + n
    ec = epad // (NW * CHUNK)
    src_e = jnp.concatenate([src, pad_i]).reshape(NW, ec, 1, CHUNK)
    dst_e = jnp.concatenate([dst, pad_i]).reshape(NW, ec, 1, CHUNK)
    idx_e = jnp.concatenate([src_e, dst_e], axis=2)

    bpad = jnp.pad(batch.astype(jnp.int32), (0, npad - n),
                   constant_values=G).reshape(1, npad)

    zer_n = jnp.zeros((npad // NS, D), jnp.float32)
    zer_nh = jnp.zeros((npad // NS, LANES), jnp.float32)
    ones_h = jnp.ones((CHUNK, LANES), jnp.float32)

    xp = jnp.pad(x.astype(jnp.int32), ((0, npad - n), (0, 16 - x.shape[1])))
    tcat = jnp.concatenate(atom_embs, axis=0)
    tcat = jnp.pad(tcat, ((0, TCAT_ROWS - tcat.shape[0]), (0, 0)))

    td = _hist_call(dst_e.reshape(NW, ec, CHUNK), ones_h, zer_nh, npad)

    g = pl.pallas_call(
        _enc_body,
        grid=(nblk,),
        in_specs=[
            pl.BlockSpec((ROWS_BLK, 16), lambda i: (i, 0)),
            pl.BlockSpec((TCAT_ROWS, D), lambda i: (0, 0)),
            pl.BlockSpec((2, ROWS_BLK, LANES), lambda i: (0, i, 0)),
        ],
        out_specs=pl.BlockSpec((ROWS_BLK, D), lambda i: (i, 0)),
        out_shape=jax.ShapeDtypeStruct((npad, D), jnp.float32),
    )(xp, tcat, td)

    for l in range(len(Ws)):
        t = _segsum_call(g, idx_e, zer_n, npad)
        g = pl.pallas_call(
            functools.partial(_layer_body, last=(l == len(Ws) - 1)),
            grid=(nblk,),
            in_specs=[
                pl.BlockSpec((2, ROWS_BLK, D), lambda i: (0, i, 0)),
                pl.BlockSpec((ROWS_BLK, D), lambda i: (i, 0)),
                pl.BlockSpec((2, ROWS_BLK, LANES), lambda i: (0, i, 0)),
                pl.BlockSpec((D, D), lambda i: (0, 0)),
                pl.BlockSpec((1, D), lambda i: (0, 0)),
            ],
            out_specs=pl.BlockSpec((ROWS_BLK, D), lambda i: (i, 0)),
            out_shape=jax.ShapeDtypeStruct((npad, D), jnp.float32),
        )(t, g, td, Ws[l], bs[l].reshape(1, D))

    z = pl.pallas_call(
        _final_body,
        grid=(1,),
        in_specs=[
            pl.BlockSpec((npad, D), lambda i: (0, 0)),
            pl.BlockSpec((1, npad), lambda i: (0, 0)),
            pl.BlockSpec((D, D), lambda i: (0, 0)),
            pl.BlockSpec((1, D), lambda i: (0, 0)),
            pl.BlockSpec((1, D), lambda i: (0, 0)),
            pl.BlockSpec((1, D), lambda i: (0, 0)),
        ],
        out_specs=pl.BlockSpec((G, D), lambda i: (0, 0)),
        out_shape=jax.ShapeDtypeStruct((G, D), jnp.float32),
    )(g, bpad, fc_W, fc_b.reshape(1, D), ln_g.reshape(1, D), ln_b.reshape(1, D))
    return z
